# baseline (speedup 1.0000x reference)
import jax
import jax.numpy as jnp
from jax import lax
from jax.experimental import pallas as pl
from jax.experimental.pallas import tpu as pltpu
from jax.experimental.pallas import tpu_sc as plsc

_B = 2
_NC = 2
_CF = 512
_HM, _WM = 32, 64
_HF, _WF = 128, 256
_N = 64
_KN = 192
_NB = 48
_NCOV = _N - _NB
_L = 16
_NW = 32
_PPW = _B * _N // _NW
_CSTRIDE = _HF * _WF
_BSTRIDE = _CF * _CSTRIDE

_TC_OUT_SHAPES = [
    jax.ShapeDtypeStruct((_B, _N, 2), jnp.float32),
    jax.ShapeDtypeStruct((_B * _N, 16, 128), jnp.int32),
    jax.ShapeDtypeStruct((_B * _N, 4), jnp.float32),
    jax.ShapeDtypeStruct((_B * _N, _NC), jnp.float32),
]


def _corner_meta(px, py, H, W):
    gx = 2.0 * px - 1.0
    gy = 2.0 * py - 1.0
    fx = ((gx + 1.0) * W - 1.0) / 2.0
    fy = ((gy + 1.0) * H - 1.0) / 2.0
    x0 = jnp.floor(fx)
    y0 = jnp.floor(fy)
    x1 = x0 + 1.0
    y1 = y0 + 1.0
    wx1 = fx - x0
    wx0 = 1.0 - wx1
    wy1 = fy - y0
    wy0 = 1.0 - wy1

    def meta(xx, yy):
        valid = (xx >= 0) & (xx <= W - 1) & (yy >= 0) & (yy <= H - 1)
        ix = jnp.clip(xx, 0, W - 1).astype(jnp.int32)
        iy = jnp.clip(yy, 0, H - 1).astype(jnp.int32)
        return valid, ix, iy

    corners = [meta(x0, y0), meta(x1, y0), meta(x0, y1), meta(x1, y1)]
    weights = [wx0 * wy0, wx1 * wy0, wx0 * wy1, wx1 * wy1]
    return corners, weights


def _gather1(flat_row, iy, ix, valid, W):
    P = iy.shape[0]
    idx = iy * W + ix
    j = lax.broadcasted_iota(jnp.int32, (P, flat_row.shape[1]), 1)
    picked = jnp.sum(jnp.where(j == idx, flat_row, 0.0), axis=1, keepdims=True)
    return picked * valid.astype(jnp.float32)


def _tc_body(mask_ref, over_ref, cov_ref, w_ref, b_ref,
             pts_ref, idx_ref, wgt_ref, coar_ref):
    mask = mask_ref[...]
    over = over_ref[...]
    cov = cov_ref[...]

    bases_b, wgts_b, rc_b = [], [], []
    pts_all = []
    for b in range(_B):
        ox = over[b][:, 0:1]
        oy = over[b][:, 1:2]
        corners, weights = _corner_meta(ox, oy, _HM, _WM)
        og = []
        for c in range(_NC):
            row = mask[2 * b + c : 2 * b + c + 1, :]
            acc = None
            for (valid, ix, iy), wgt in zip(corners, weights):
                term = _gather1(row, iy, ix, valid, _WM) * wgt
                acc = term if acc is None else acc + term
            og.append(acc)
        hi = jnp.maximum(og[0], og[1])
        lo = jnp.minimum(og[0], og[1])
        unc = -1.0 * (hi - lo)

        unc_t = jnp.reshape(unc, (1, _KN))
        gt = unc_t > unc
        eq = unc_t == unc
        jlt = (lax.broadcasted_iota(jnp.int32, (_KN, _KN), 1)
               < lax.broadcasted_iota(jnp.int32, (_KN, _KN), 0))
        rank = jnp.sum((gt | (eq & jlt)).astype(jnp.int32), axis=1,
                       keepdims=True)

        r_i = lax.broadcasted_iota(jnp.int32, (_N, _KN), 0)
        sel = (r_i == jnp.reshape(rank, (1, _KN))) & (r_i < _NB)
        px = jnp.sum(jnp.where(sel, jnp.reshape(ox, (1, _KN)), 0.0),
                     axis=1, keepdims=True)
        py = jnp.sum(jnp.where(sel, jnp.reshape(oy, (1, _KN)), 0.0),
                     axis=1, keepdims=True)
        c_i = lax.broadcasted_iota(jnp.int32, (_N, _NCOV), 1)
        r_v = lax.broadcasted_iota(jnp.int32, (_N, _NCOV), 0) - _NB
        selc = r_v == c_i
        px = px + jnp.sum(jnp.where(selc, jnp.reshape(cov[b][:, 0:1], (1, _NCOV)), 0.0),
                          axis=1, keepdims=True)
        py = py + jnp.sum(jnp.where(selc, jnp.reshape(cov[b][:, 1:2], (1, _NCOV)), 0.0),
                          axis=1, keepdims=True)
        pts_all.append(jnp.concatenate([px, py], axis=1))

        pc_corners, pc_weights = _corner_meta(px, py, _HM, _WM)
        gch = []
        for c in range(_NC):
            row = mask[2 * b + c : 2 * b + c + 1, :]
            acc = None
            for (valid, ix, iy), wgt in zip(pc_corners, pc_weights):
                term = _gather1(row, iy, ix, valid, _WM) * wgt
                acc = term if acc is None else acc + term
            gch.append(acc)
        rc = []
        for o in range(_NC):
            rc.append(gch[0] * w_ref[o, 0] + gch[1] * w_ref[o, 1] + b_ref[0, o])
        rc_b.append(jnp.concatenate(rc, axis=1))

        f_corners, f_weights = _corner_meta(px, py, _HF, _WF)
        bs, ws = [], []
        for (valid, ix, iy), wgt in zip(f_corners, f_weights):
            bs.append(b * _BSTRIDE + iy * _WF + ix)
            ws.append(wgt * valid.astype(jnp.float32))
        bases_b.append(jnp.concatenate(bs, axis=1))
        wgts_b.append(jnp.concatenate(ws, axis=1))

    bases = jnp.concatenate(bases_b, axis=0)
    wgts = jnp.concatenate(wgts_b, axis=0)

    pts_ref[...] = jnp.stack(pts_all, axis=0)
    wgt_ref[...] = wgts

    b3 = jnp.concatenate(
        [jnp.broadcast_to(bases[:, k : k + 1], (_B * _N, 4)) for k in range(4)],
        axis=1)
    kj = lax.broadcasted_iota(jnp.int32, (_B * _N, 16, 128), 1)
    cc = lax.broadcasted_iota(jnp.int32, (_B * _N, 16, 128), 2)
    idx_ref[...] = b3[:, :, None] + ((kj % 4) * 128 + cc) * _CSTRIDE

    coar_ref[...] = jnp.concatenate(rc_b, axis=0)


def _sc_body(table, idxr, wgt16, coar16, wf, out16,
             idx_all, rows_a, rows_b, wgt_v, acc_v, w0_v, w1_v,
             sem_a, sem_b):
    wid = lax.axis_index("s") * 2 + lax.axis_index("c")
    pltpu.sync_copy(wf.at[0], w0_v)
    pltpu.sync_copy(wf.at[1], w1_v)
    pltpu.sync_copy(wgt16.at[wid], wgt_v)
    pltpu.sync_copy(coar16.at[wid], acc_v)
    pltpu.sync_copy(idxr.at[pl.ds(wid * _PPW, _PPW)], idx_all)

    rows = [rows_a, rows_b]
    sems = [sem_a, sem_b]

    def issue(i, buf):
        p, k = i // 4, i % 4
        return [
            pltpu.async_copy(table.at[idx_all.at[p, 4 * k + j]],
                             rows[buf].at[pl.ds(j * 128, 128)], sems[buf])
            for j in range(4)
        ]

    pend = [None, None]
    pend[0] = issue(0, 0)
    iota16 = lax.iota(jnp.int32, _L)
    zeros16 = jnp.zeros((_L,), jnp.float32)
    row_acc = acc_v[...]
    wgts_vec = wgt_v[...]
    for i in range(_PPW * 4):
        buf = i % 2
        if i + 1 < _PPW * 4:
            pend[1 - buf] = issue(i + 1, 1 - buf)
        for d in pend[buf]:
            d.wait()
        w_s = wgts_vec[i]
        rbuf = rows[buf]

        def chunk(kk, accs, rbuf=rbuf):
            a0, a1 = accs
            col = rbuf[pl.ds(kk * _L, _L)]
            a0 = a0 + col * w0_v[pl.ds(kk * _L, _L)]
            a1 = a1 + col * w1_v[pl.ds(kk * _L, _L)]
            return (a0, a1)

        a0, a1 = lax.fori_loop(0, _CF // _L, chunk, (zeros16, zeros16))
        t0 = jnp.sum(a0) * w_s
        t1 = jnp.sum(a1) * w_s
        slot = (i // 4) * 2
        row_acc = (row_acc
                   + jnp.where(iota16 == slot, jnp.full((_L,), t0), zeros16)
                   + jnp.where(iota16 == slot + 1, jnp.full((_L,), t1), zeros16))
    acc_v[...] = row_acc
    pltpu.sync_copy(acc_v, out16.at[wid])


def _sc_call(table, idx3, wgt16, coar16, wf):
    mesh = plsc.VectorSubcoreMesh(core_axis_name="c", subcore_axis_name="s")
    return pl.kernel(
        _sc_body,
        out_type=jax.ShapeDtypeStruct((_NW, _L), jnp.float32),
        mesh=mesh,
        compiler_params=pltpu.CompilerParams(needs_layout_passes=False),
        scratch_types=[
            pltpu.VMEM((_PPW, 16, 128), jnp.int32),
            pltpu.VMEM((_CF,), jnp.float32),
            pltpu.VMEM((_CF,), jnp.float32),
            pltpu.VMEM((_L,), jnp.float32),
            pltpu.VMEM((_L,), jnp.float32),
            pltpu.VMEM((_CF,), jnp.float32),
            pltpu.VMEM((_CF,), jnp.float32),
            pltpu.SemaphoreType.DMA,
            pltpu.SemaphoreType.DMA,
        ],
    )(table, idx3, wgt16, coar16, wf)


def kernel(x, res2, out, W, b):
    del x
    rng = jax.random.key(42)
    r1, r2 = jax.random.split(rng)
    over = jax.random.uniform(r1, (_B, _KN, 2), dtype=jnp.float32)
    coverage = jax.random.uniform(r2, (_B, _NCOV, 2), dtype=jnp.float32)

    mask_flat = out.reshape(_B * _NC, _HM * _WM)
    points, idx3, wgts, rc = pl.pallas_call(
        _tc_body, out_shape=_TC_OUT_SHAPES,
    )(mask_flat, over, coverage, W, b.reshape(1, _NC))
    wgt16 = wgts.reshape(_NW, _L)
    coar16 = jnp.concatenate(
        [rc.reshape(_NW, 8), jnp.zeros((_NW, 8), jnp.float32)], axis=1)

    table = res2.reshape(-1)
    wf = W[:, _NC:]
    out16 = _sc_call(table, idx3, wgt16, coar16, wf)

    rend = out16[:, :8].reshape(_B, _N, _NC).transpose(0, 2, 1)
    return rend, points
